# Initial kernel scaffold; baseline (speedup 1.0000x reference)
import functools

import jax
import jax.numpy as jnp
from jax import lax
from jax.experimental import pallas as pl
from jax.experimental.pallas import tpu as pltpu
from jax.experimental.pallas import tpu_sc as plsc

_N = 16
_C = 68
_H = 128
_W = 128
_HW = _H * _W
_K = 9
_L = 16
_NW = 32
_ROWS = _N * _C
_RPW = _ROWS // _NW
_GROUP = 4
_NGROUPS = _HW // (_GROUP * _L)


def _row_topk_coord(buf, j, outbuf):
    iota = lax.iota(jnp.int32, 16)
    ninf = jnp.full((_L,), -jnp.inf, jnp.float32)

    def group_body(g, carry):
        t_val, t_idx, t_min = carry
        base = g * (_GROUP * _L)
        vs = [buf[pl.ds(base + k * _L, _L)] for k in range(_GROUP)]
        vm = jnp.maximum(jnp.maximum(vs[0], vs[1]), jnp.maximum(vs[2], vs[3]))
        pred = jnp.any(vm > t_min)

        def slow(tv, ti, tm):
            for k in range(_GROUP):
                def merge(tv, ti, tm, _v=vs[k], _off=base + k * _L):
                    cidx = iota + _off
                    cv, ci = plsc.sort_key_val(_v, cidx)
                    cv = lax.rev(cv, (0,))
                    ci = lax.rev(ci, (0,))
                    take = cv > tv
                    hv = jnp.where(take, cv, tv)
                    hi = jnp.where(take, ci, ti)
                    tv2, ti2 = plsc.sort_key_val(hv, hi)
                    return tv2, ti2, jnp.full((_L,), jnp.min(tv2))

                pk = jnp.any(vs[k] > tm)
                tv, ti, tm = lax.cond(pk, merge, lambda a, b, c: (a, b, c),
                                      tv, ti, tm)
            return tv, ti, tm

        return lax.cond(pred, slow, lambda a, b, c: (a, b, c),
                        t_val, t_idx, t_min)

    t_val, t_idx, _ = lax.fori_loop(
        0, _NGROUPS, group_body,
        (ninf, jnp.zeros((_L,), jnp.int32), ninf))

    srt_val, _ = plsc.sort_key_val(t_val, t_idx, descending=True)
    v9 = jnp.max(jnp.where(iota == (_K - 1), srt_val, ninf))
    v9s = jnp.full((_L,), v9)
    gt = t_val > v9s
    cnt_gt = plsc.all_reduce_population_count(gt)
    eq = t_val == v9s
    key_idx = jnp.where(eq, t_idx, jnp.int32(1 << 30))
    sidx, _ = plsc.sort_key_val(key_idx, key_idx)
    rlane = jnp.full((_L,), _K - 1, jnp.int32) - cnt_gt
    idx_thr = jnp.max(jnp.where(iota == rlane, sidx, jnp.int32(-(1 << 30))))
    chosen = gt | (eq & (t_idx <= jnp.full((_L,), idx_thr)))

    smax = jnp.max(t_val)
    w = jnp.where(chosen, jnp.exp(t_val - jnp.full((_L,), smax)),
                  jnp.float32(0.0))
    den = jnp.sum(w)
    xf = (t_idx & (_W - 1)).astype(jnp.float32)
    yf = (t_idx >> 7).astype(jnp.float32)
    nx = jnp.sum(w * xf)
    ny = jnp.sum(w * yf)
    ox = jnp.full((_L,), nx) * 4.0 / jnp.full((_L,), den)
    oy = jnp.full((_L,), ny) * 4.0 / jnp.full((_L,), den)
    res = jnp.where(iota == 0, ox, jnp.where(iota == 1, oy, jnp.float32(0.0)))
    outbuf[pl.ds(j * _L, _L)] = res


def _make_sc_call():
    mesh = plsc.VectorSubcoreMesh(core_axis_name="c", subcore_axis_name="s")

    @functools.partial(
        pl.kernel,
        out_type=jax.ShapeDtypeStruct((_NW, _RPW * _L), jnp.float32),
        mesh=mesh,
        scratch_types=[
            pltpu.VMEM((_HW,), jnp.float32),
            pltpu.VMEM((_RPW * _L,), jnp.float32),
        ],
    )
    def sc_kernel(x_hbm, out_hbm, buf, outbuf):
        wid = lax.axis_index("s") * 2 + lax.axis_index("c")

        def row_loop(j, carry):
            r = wid * _RPW + j
            n = r // _C
            c = r % _C
            hrow = n * (2 * _C) + _C + c
            pltpu.sync_copy(x_hbm.at[hrow], buf)
            _row_topk_coord(buf, j, outbuf)
            return carry

        lax.fori_loop(0, _RPW, row_loop, 0)
        pltpu.sync_copy(outbuf, out_hbm.at[wid])

    return sc_kernel


_sc_call = _make_sc_call()


@jax.jit
def kernel(input):
    x = input.reshape(_N * 2 * _C, _HW)
    out = _sc_call(x)
    out = out.reshape(_ROWS, _L)[:, :2]
    return out.reshape(_N, _C, 2)


# speedup vs baseline: 13.1373x; 13.1373x over previous
import functools

import jax
import jax.numpy as jnp
from jax import lax
from jax.experimental import pallas as pl
from jax.experimental.pallas import tpu as pltpu
from jax.experimental.pallas import tpu_sc as plsc

_N = 16
_C = 68
_H = 128
_W = 128
_HW = _H * _W
_K = 9
_L = 16
_NW = 32
_ROWS = _N * _C
_RPW = _ROWS // _NW
_GROUP = 4
_NGROUPS = _HW // (_GROUP * _L)


def _row_topk_coord(buf, j, outbuf):
    iota = lax.iota(jnp.int32, 16)
    ninf = jnp.full((_L,), -jnp.inf, jnp.float32)

    def group_body(g, carry):
        t_val, t_idx, t_min = carry
        base = g * (_GROUP * _L)
        vs = [buf[pl.ds(base + k * _L, _L)] for k in range(_GROUP)]
        vm = jnp.maximum(jnp.maximum(vs[0], vs[1]), jnp.maximum(vs[2], vs[3]))
        pred = jnp.any(vm > t_min)

        def slow(tv, ti, tm):
            for k in range(_GROUP):
                def merge(tv, ti, tm, _v=vs[k], _off=base + k * _L):
                    cidx = iota + _off
                    cv, ci = plsc.sort_key_val(_v, cidx)
                    cv = lax.rev(cv, (0,))
                    ci = lax.rev(ci, (0,))
                    take = cv > tv
                    hv = jnp.where(take, cv, tv)
                    hi = jnp.where(take, ci, ti)
                    tv2, ti2 = plsc.sort_key_val(hv, hi)
                    return tv2, ti2, jnp.full((_L,), jnp.min(tv2))

                pk = jnp.any(vs[k] > tm)
                tv, ti, tm = lax.cond(pk, merge, lambda a, b, c: (a, b, c),
                                      tv, ti, tm)
            return tv, ti, tm

        return lax.cond(pred, slow, lambda a, b, c: (a, b, c),
                        t_val, t_idx, t_min)

    t_val, t_idx, _ = lax.fori_loop(
        0, _NGROUPS, group_body,
        (ninf, jnp.zeros((_L,), jnp.int32), ninf))

    srt_val, _ = plsc.sort_key_val(t_val, t_idx, descending=True)
    v9 = jnp.max(jnp.where(iota == (_K - 1), srt_val, ninf))
    v9s = jnp.full((_L,), v9)
    gt = t_val > v9s
    cnt_gt = plsc.all_reduce_population_count(gt)
    eq = t_val == v9s
    key_idx = jnp.where(eq, t_idx, jnp.int32(1 << 30))
    sidx, _ = plsc.sort_key_val(key_idx, key_idx)
    rlane = jnp.full((_L,), _K - 1, jnp.int32) - cnt_gt
    idx_thr = jnp.max(jnp.where(iota == rlane, sidx, jnp.int32(-(1 << 30))))
    chosen = gt | (eq & (t_idx <= jnp.full((_L,), idx_thr)))

    smax = jnp.max(t_val)
    w = jnp.where(chosen, jnp.exp(t_val - jnp.full((_L,), smax)),
                  jnp.float32(0.0))
    den = jnp.sum(w)
    xf = (t_idx & (_W - 1)).astype(jnp.float32)
    yf = (t_idx >> 7).astype(jnp.float32)
    nx = jnp.sum(w * xf)
    ny = jnp.sum(w * yf)
    ox = jnp.full((_L,), nx) * 4.0 / jnp.full((_L,), den)
    oy = jnp.full((_L,), ny) * 4.0 / jnp.full((_L,), den)
    res = jnp.where(iota == 0, ox, jnp.where(iota == 1, oy, jnp.float32(0.0)))
    outbuf[pl.ds(j * _L, _L)] = res


def _make_sc_call():
    mesh = plsc.VectorSubcoreMesh(core_axis_name="c", subcore_axis_name="s")

    @functools.partial(
        pl.kernel,
        out_type=jax.ShapeDtypeStruct((_NW, _RPW * _L), jnp.float32),
        mesh=mesh,
        scratch_types=[
            pltpu.VMEM((_HW,), jnp.float32),
            pltpu.VMEM((_RPW * _L,), jnp.float32),
        ],
        compiler_params=pltpu.CompilerParams(needs_layout_passes=False),
    )
    def sc_kernel(x_hbm, out_hbm, buf, outbuf):
        wid = lax.axis_index("s") * 2 + lax.axis_index("c")

        def row_loop(j, carry):
            r = wid * _RPW + j
            n = r // _C
            c = r % _C
            hrow = n * (2 * _C) + _C + c
            pltpu.sync_copy(x_hbm.at[hrow], buf)
            _row_topk_coord(buf, j, outbuf)
            return carry

        lax.fori_loop(0, _RPW, row_loop, 0)
        pltpu.sync_copy(outbuf, out_hbm.at[wid])

    return sc_kernel


_sc_call = _make_sc_call()


@jax.jit
def kernel(input):
    x = input.reshape(_N * 2 * _C, _HW)
    out = _sc_call(x)
    out = out.reshape(_ROWS, _L)[:, :2]
    return out.reshape(_N, _C, 2)


# speedup vs baseline: 47.9798x; 3.6522x over previous
import functools

import jax
import jax.numpy as jnp
from jax import lax
from jax.experimental import pallas as pl
from jax.experimental.pallas import tpu as pltpu
from jax.experimental.pallas import tpu_sc as plsc

_N = 16
_C = 68
_H = 128
_W = 128
_HW = _H * _W
_K = 9
_L = 16
_NW = 32
_ROWS = _N * _C
_RPW = _ROWS // _NW
_NACC = 4
_Q = _HW // _NACC
_BPB = _Q // _L
_BIG = 1 << 30


def _splat(x):
    return jnp.full((_L,), x)


def _row_topk_coord(buf, j, outbuf):
    iota = lax.iota(jnp.int32, _L)
    viota16 = iota * 16
    ninf_v = _splat(jnp.float32(-jnp.inf))
    zero_i = jnp.zeros((_L,), jnp.int32)

    def fold_body(b, carry):
        avs, ais = carry
        off0 = b * 16
        new_avs, new_ais = [], []
        for t in range(_NACC):
            off = t * _Q + off0
            v = buf[pl.ds(off, _L)]
            idxv = iota + off
            gt = v > avs[t]
            new_avs.append(jnp.where(gt, v, avs[t]))
            new_ais.append(jnp.where(gt, idxv, ais[t]))
        return tuple(new_avs), tuple(new_ais)

    avs, ais = lax.fori_loop(
        0, _BPB, fold_body,
        ((ninf_v,) * _NACC, (zero_i,) * _NACC))

    def ext_body(k, carry):
        av0, av1, av2, av3, ai0, ai1, ai2, ai3, s_vec, i_vec = carry
        avs = [av0, av1, av2, av3]
        ais = [ai0, ai1, ai2, ai3]
        mm = jnp.maximum(jnp.maximum(avs[0], avs[1]),
                         jnp.maximum(avs[2], avs[3]))
        m = _splat(jnp.max(mm))
        cand = jnp.minimum(
            jnp.minimum(jnp.where(avs[0] == m, ais[0], _BIG),
                        jnp.where(avs[1] == m, ais[1], _BIG)),
            jnp.minimum(jnp.where(avs[2] == m, ais[2], _BIG),
                        jnp.where(avs[3] == m, ais[3], _BIG)))
        wi = _splat(jnp.min(cand))
        ksplat = _splat(k)
        s_vec = jnp.where(iota == ksplat, m, s_vec)
        i_vec = jnp.where(iota == ksplat, wi, i_vec)
        plsc.store_scatter(buf, [wi], ninf_v, mask=iota == 0)
        base = (wi & 15) + (wi & (3 << 12))
        idxj = base + viota16
        rv, ri = ninf_v, zero_i
        for _ in range(_BPB // _L):
            g = plsc.load_gather(buf, [idxj])
            gt = g > rv
            rv = jnp.where(gt, g, rv)
            ri = jnp.where(gt, idxj, ri)
            idxj = idxj + 256
        m2 = _splat(jnp.max(rv))
        w2 = _splat(jnp.min(jnp.where(rv == m2, ri, _BIG)))
        is_lane = iota == (wi & 15)
        tq = wi >> 12
        for t in range(_NACC):
            hit = is_lane & (tq == t)
            avs[t] = jnp.where(hit, m2, avs[t])
            ais[t] = jnp.where(hit, w2, ais[t])
        return (avs[0], avs[1], avs[2], avs[3],
                ais[0], ais[1], ais[2], ais[3], s_vec, i_vec)

    carry = (avs[0], avs[1], avs[2], avs[3],
             ais[0], ais[1], ais[2], ais[3], ninf_v, zero_i)
    carry = lax.fori_loop(0, _K, ext_body, carry)
    s_vec, i_vec = carry[8], carry[9]

    smax = _splat(jnp.max(s_vec))
    w = jnp.where(iota < _K, jnp.exp(s_vec - smax), jnp.float32(0.0))
    den = jnp.sum(w)
    xf = (i_vec & (_W - 1)).astype(jnp.float32)
    yf = (i_vec >> 7).astype(jnp.float32)
    nx = jnp.sum(w * xf)
    ny = jnp.sum(w * yf)
    ox = _splat(nx) * 4.0 / _splat(den)
    oy = _splat(ny) * 4.0 / _splat(den)
    res = jnp.where(iota == 0, ox, jnp.where(iota == 1, oy, jnp.float32(0.0)))
    outbuf[pl.ds(j * _L, _L)] = res


def _make_sc_call():
    mesh = plsc.VectorSubcoreMesh(core_axis_name="c", subcore_axis_name="s")

    @functools.partial(
        pl.kernel,
        out_type=jax.ShapeDtypeStruct((_NW, _RPW * _L), jnp.float32),
        mesh=mesh,
        scratch_types=[
            pltpu.VMEM((_HW,), jnp.float32),
            pltpu.VMEM((_HW,), jnp.float32),
            pltpu.VMEM((_RPW * _L,), jnp.float32),
            pltpu.SemaphoreType.DMA,
            pltpu.SemaphoreType.DMA,
        ],
        compiler_params=pltpu.CompilerParams(needs_layout_passes=False),
    )
    def sc_kernel(x_hbm, out_hbm, buf0, buf1, outbuf, sem0, sem1):
        wid = lax.axis_index("s") * 2 + lax.axis_index("c")

        def hrow(j):
            r = wid * _RPW + j
            n = r // _C
            c = r % _C
            return n * (2 * _C) + _C + c

        pltpu.async_copy(x_hbm.at[hrow(0)], buf0, sem0)

        def pair_body(p, carry):
            j0 = p * 2
            pltpu.async_copy(x_hbm.at[hrow(j0 + 1)], buf1, sem1)
            pltpu.make_async_copy(x_hbm.at[hrow(j0)], buf0, sem0).wait()
            _row_topk_coord(buf0, j0, outbuf)

            @pl.when(p < _RPW // 2 - 1)
            def _():
                pltpu.async_copy(x_hbm.at[hrow(j0 + 2)], buf0, sem0)

            pltpu.make_async_copy(x_hbm.at[hrow(j0 + 1)], buf1, sem1).wait()
            _row_topk_coord(buf1, j0 + 1, outbuf)
            return carry

        lax.fori_loop(0, _RPW // 2, pair_body, 0)
        pltpu.sync_copy(outbuf, out_hbm.at[wid])

    return sc_kernel


_sc_call = _make_sc_call()


@jax.jit
def kernel(input):
    x = input.reshape(_N * 2 * _C, _HW)
    out = _sc_call(x)
    out = out.reshape(_ROWS, _L)[:, :2]
    return out.reshape(_N, _C, 2)
